# baseline (speedup 1.0000x reference)
import functools

import jax
import jax.numpy as jnp
from jax import lax
from jax.experimental import pallas as pl
from jax.experimental.pallas import tpu as pltpu
from jax.experimental.pallas import tpu_sc as plsc

_S = 50
_SB = 10
_NSB = _S // _SB
_L = 128
_Q = 4
_I = 8
_D = 32
_CH = _L * _SB


@functools.lru_cache(maxsize=None)
def _build(NB: int, V: int):
    B = NB * _S
    info = plsc.get_sparse_core_info()
    NC, NS = info.num_cores, info.num_subcores
    NW = NC * NS
    J = NB // _L
    JW = J // NW
    b_per_w = B // NW
    SB = 5
    CH = _L * SB
    NCH = JW * (_S // SB)

    mesh = plsc.VectorSubcoreMesh(core_axis_name="c", subcore_axis_name="s")

    @functools.partial(
        pl.kernel,
        mesh=mesh,
        out_type=jax.ShapeDtypeStruct((_S, _Q, J, _I, _L), jnp.float32),
        scratch_types=[
            pltpu.VMEM((b_per_w,), jnp.int32),
            pltpu.VMEM((CH,), jnp.int32),
            pltpu.VMEM((CH,), jnp.int32),
            pltpu.VMEM((CH, _D), jnp.float32),
            pltpu.VMEM((CH, _D), jnp.float32),
            pltpu.VMEM((SB, _Q, 1, _I, _L + 1), jnp.float32),
            pltpu.VMEM((SB, _Q, 1, _I, _L + 1), jnp.float32),
            pltpu.SemaphoreType.DMA,
            pltpu.SemaphoreType.DMA,
            pltpu.SemaphoreType.DMA,
            pltpu.SemaphoreType.DMA,
        ],
        compiler_params=pltpu.CompilerParams(
            use_tc_tiling_on_sc=False, needs_layout_passes=False),
    )
    def gather_kernel(idx_hbm, table_hbm, out_hbm, idx_all, ic_a, ic_b,
                      r_a, r_b, ot_a, ot_b, g_a, g_b, o_a, o_b):
        wid = lax.axis_index("s") * NC + lax.axis_index("c")
        base = wid * b_per_w
        pltpu.sync_copy(idx_hbm.at[pl.ds(base, b_per_w)], idx_all)

        iota = lax.iota(jnp.int32, 16)
        iota_sb = iota * SB
        iota_s = iota * _S
        vzero = lax.broadcast(jnp.int32(0), (16,))

        def idx_build(cidx, ic):
            jj = lax.bitwise_and(cidx, JW - 1)
            sb = lax.shift_right_logical(cidx, 2)
            cbase = jj * _L * _S + sb * SB
            for ss in range(SB):
                for bg in range(_L // 16):
                    vpos = iota_s + (cbase + bg * 16 * _S + ss)
                    vals = plsc.load_gather(idx_all, [vpos])
                    vdst = iota_sb + (bg * 16 * SB + ss)
                    plsc.store_scatter(ic, [vdst], vals)

        def gather(ic, r, sem):
            return pltpu.make_async_copy(table_hbm.at[ic], r, sem)

        def transpose(r, ot):
            @pl.loop(0, _L)
            def _(bb):
                vbb = lax.broadcast(bb, (16,))
                for ss in range(SB):
                    k = bb * SB + ss
                    vss = lax.broadcast(jnp.int32(ss), (16,))
                    for cg in range(_D // 16):
                        vals = r[k, pl.ds(cg * 16, 16)]
                        vc = iota + cg * 16
                        vq = lax.shift_right_logical(vc, 3)
                        vi = lax.bitwise_and(vc, 7)
                        plsc.store_scatter(
                            ot, [vss, vq, vzero, vi, vbb], vals)

        def out_dma(cidx, ot, sem):
            jj = lax.bitwise_and(cidx, JW - 1)
            sb = lax.shift_right_logical(cidx, 2)
            return pltpu.make_async_copy(
                ot.at[:, :, :, :, pl.ds(0, _L)],
                out_hbm.at[pl.ds(sb * SB, SB), :,
                           pl.ds(wid * JW + jj, 1), :, :], sem)

        idx_build(0, ic_a)
        gather(ic_a, r_a, g_a).start()

        @pl.loop(0, NCH, step=2)
        def _(i):
            idx_build(i + 1, ic_b)
            gather(ic_b, r_b, g_b).start()
            gather(ic_a, r_a, g_a).wait()

            @pl.when(i > 0)
            def _():
                out_dma(i - 2, ot_a, o_a).wait()
            transpose(r_a, ot_a)
            out_dma(i, ot_a, o_a).start()

            gather(ic_b, r_b, g_b).wait()

            @pl.when(i + 2 < NCH)
            def _():
                idx_build(i + 2, ic_a)
                gather(ic_a, r_a, g_a).start()

            @pl.when(i > 0)
            def _():
                out_dma(i - 1, ot_b, o_b).wait()
            transpose(r_b, ot_b)
            out_dma(i + 1, ot_b, o_b).start()

        out_dma(NCH - 2, ot_a, o_a).wait()
        out_dma(NCH - 1, ot_b, o_b).wait()

    return gather_kernel


def kernel(x, weight):
    NB, S = x.shape
    V, D = weight.shape
    flat = x.reshape(NB * S).astype(jnp.int32)
    out5 = _build(NB, V)(flat, weight)
    return jnp.transpose(out5, (2, 4, 0, 1, 3)).reshape(NB, S, D)
